# baseline (speedup 1.0000x reference)
import functools

import jax
import jax.numpy as jnp
from jax import lax
from jax.experimental import pallas as pl
from jax.experimental.pallas import tpu as pltpu
from jax.experimental.pallas import tpu_sc as plsc

_D = 128
_T = 20
_B = 32
_S = 5
_H = 50
_ROWS = 40
_NB = 1760 // _ROWS
_CBLK = 160 // _ROWS

_K = 10
_NW = 32
_N_NODES = 35200
_WPN = 1104
_N_PAD = _NW * _WPN
_C = 24
_SUB = _WPN // _C


def _elu(x):
    return jnp.where(x > 0, x, jnp.exp(x) - 1.0)


def _sc_body(nodes_hbm, flate_hbm, flatr_hbm, ent_hbm, rel_hbm,
             node_out, agg_out,
             idx_v, flate_v, flatr_v, node_a, node_b, nb_v, agg_a, agg_b,
             rel_v, sem0, sem1, sem2, semw0, semw1):
    wid = lax.axis_index("s") * 2 + lax.axis_index("c")
    slab = wid * _WPN
    pltpu.sync_copy(rel_hbm, rel_v)
    pltpu.sync_copy(nodes_hbm.at[pl.ds(slab, _WPN)], idx_v)
    pltpu.sync_copy(flate_hbm.at[pl.ds(slab * _K, _WPN * _K)], flate_v)
    pltpu.sync_copy(flatr_hbm.at[pl.ds(slab * _K, _WPN * _K)],
                    flatr_v.at[pl.ds(0, _WPN * _K)])

    _HC = _C // 2
    _HR = _HC * _K

    def start_half(g, buf):
        return pltpu.async_copy(
            ent_hbm.at[flate_v.at[pl.ds(g * _HR, _HR)]],
            nb_v.at[pl.ds(buf * _HR, _HR)], sem0 if buf == 0 else sem1)

    def wait_half(buf):
        pltpu.make_async_copy(
            ent_hbm.at[flate_v.at[pl.ds(0, _HR)]],
            nb_v.at[pl.ds(buf * _HR, _HR)], sem0 if buf == 0 else sem1).wait()

    def drain_writes(node_buf, agg_buf, semw):
        pltpu.make_async_copy(node_buf, node_out.at[pl.ds(0, _C)],
                              semw).wait()
        pltpu.make_async_copy(agg_buf, agg_out.at[pl.ds(0, _C)],
                              semw).wait()

    def red_half(cbase, h, agg_buf):
        def body(lu, carry2):
            for u in range(2):
                c2 = h * _HC + lu * 2 + u
                v = flatr_v[pl.ds((cbase + c2) * _K, 16)]
                rows = [v[k] for k in range(_K)]
                nbb = h * _HR + (lu * 2 + u) * _K
                for col in range(_D // 16):
                    sl = pl.ds(col * 16, 16)
                    acc = nb_v[nbb, sl]
                    for k in range(1, _K):
                        acc = acc + nb_v[nbb + k, sl]
                    for k in range(_K):
                        acc = acc + rel_v[rows[k], sl]
                    agg_buf[c2, sl] = acc
            return carry2
        lax.fori_loop(0, _HC // 2, body, 0)

    start_half(0, 0)

    def pair(p, carry):
        for par in (0, 1):
            s = 2 * p + par
            cbase = s * _C
            node_buf = node_a if par == 0 else node_b
            agg_buf = agg_a if par == 0 else agg_b
            semw = semw0 if par == 0 else semw1
            start_half(2 * s + 1, 1)

            @pl.when(p > 0)
            def _():
                drain_writes(node_buf, agg_buf, semw)

            wait_half(0)
            cp_n = pltpu.async_copy(
                ent_hbm.at[idx_v.at[pl.ds(cbase, _C)]], node_buf, sem2)
            red_half(cbase, 0, agg_buf)

            @pl.when(s < _SUB - 1)
            def _():
                start_half(2 * s + 2, 0)

            wait_half(1)
            red_half(cbase, 1, agg_buf)
            cp_n.wait()
            pltpu.async_copy(node_buf,
                             node_out.at[pl.ds(slab + cbase, _C)], semw)
            pltpu.async_copy(agg_buf,
                             agg_out.at[pl.ds(slab + cbase, _C)], semw)
        return carry

    lax.fori_loop(0, _SUB // 2, pair, 0)
    drain_writes(node_a, agg_a, semw0)
    drain_writes(node_b, agg_b, semw1)


def _sc_gather(nodes_pad, flate, flatr, ent_emb, rel_emb):
    mesh = plsc.VectorSubcoreMesh(core_axis_name="c", subcore_axis_name="s")
    f32, i32 = jnp.float32, jnp.int32
    return pl.kernel(
        _sc_body,
        out_type=(jax.ShapeDtypeStruct((_N_PAD, _D), f32),
                  jax.ShapeDtypeStruct((_N_PAD, _D), f32)),
        mesh=mesh,
        scratch_types=[
            pltpu.VMEM((_WPN,), i32),
            pltpu.VMEM((_WPN * _K,), i32),
            pltpu.VMEM((_WPN * _K + 16,), i32),
            pltpu.VMEM((_C, _D), f32),
            pltpu.VMEM((_C, _D), f32),
            pltpu.VMEM((_C * _K, _D), f32),
            pltpu.VMEM((_C, _D), f32),
            pltpu.VMEM((_C, _D), f32),
            pltpu.VMEM((500, _D), f32),
            pltpu.SemaphoreType.DMA,
            pltpu.SemaphoreType.DMA,
            pltpu.SemaphoreType.DMA,
            pltpu.SemaphoreType.DMA,
            pltpu.SemaphoreType.DMA,
        ],
    )(nodes_pad, flate, flatr, ent_emb, rel_emb)


def _dense_body(t_raw_ref, node_ref, agg_ref,
                W_c1_ref, b_c1_ref, W_c2_ref, b_c2_ref,
                W_ae_ref, b_ae_ref, W_a1_ref, b_a1_ref, W_a2_ref,
                W_m1_ref, b_m1_ref, W_m2_ref, b_m2_ref,
                out_ref, c_scr, u_scr):
    g = pl.program_id(0)

    t = t_raw_ref[...]
    t = _elu(jnp.dot(t, W_c1_ref[...], preferred_element_type=jnp.float32)
             + b_c1_ref[...])
    t = jnp.tanh(jnp.dot(t, W_c2_ref[...], preferred_element_type=jnp.float32)
                 + b_c2_ref[...])

    node = node_ref[...]
    agg = agg_ref[...]
    W_ae = W_ae_ref[...]
    a = jnp.tanh(jnp.dot(node, W_ae[:_D], preferred_element_type=jnp.float32)
                 + jnp.dot(agg, W_ae[_D:], preferred_element_type=jnp.float32)
                 + b_ae_ref[...])
    h = _elu(jnp.dot(a, W_a1_ref[...], preferred_element_type=jnp.float32)
             + b_a1_ref[...])
    w2b = jnp.broadcast_to(W_a2_ref[...], (_D, _D))
    lg = jnp.dot(h, w2b, preferred_element_type=jnp.float32)
    ex = jnp.exp(jnp.minimum(lg, 60.0))
    colg = lax.broadcasted_iota(jnp.int32, (_ROWS, _ROWS * _T), 1) // _T
    rowg = lax.broadcasted_iota(jnp.int32, (_ROWS, _ROWS * _T), 0)
    p_sum = (colg == rowg).astype(jnp.float32)
    rowg2 = lax.broadcasted_iota(jnp.int32, (_ROWS * _T, _ROWS), 0) // _T
    colg2 = lax.broadcasted_iota(jnp.int32, (_ROWS * _T, _ROWS), 1)
    p_exp = (rowg2 == colg2).astype(jnp.float32)
    s = jnp.dot(p_sum, ex, preferred_element_type=jnp.float32)
    sfull = jnp.dot(p_exp, s, preferred_element_type=jnp.float32)
    wfull = ex / (sfull + 1e-30)
    anchor = jnp.dot(p_sum, a * wfull,
                     preferred_element_type=jnp.float32)

    W_m1 = W_m1_ref[...]
    y = _elu(jnp.dot(t, W_m1[:_D], preferred_element_type=jnp.float32)
             + jnp.dot(anchor, W_m1[_D:], preferred_element_type=jnp.float32)
             + b_m1_ref[...])
    y = _elu(jnp.dot(y, W_m2_ref[...], preferred_element_type=jnp.float32)
             + b_m2_ref[...])

    @pl.when(g == 0)
    def _():
        u_scr[...] = jnp.zeros_like(u_scr)

    @pl.when(g < _CBLK)
    def _():
        c_scr[pl.ds(g * _ROWS, _ROWS), :] = y

    @pl.when(g >= _CBLK)
    def _():
        rows = (g - _CBLK) * _ROWS + lax.broadcasted_iota(jnp.int32, (_B, _ROWS), 1)
        sel = (rows // _H == lax.broadcasted_iota(jnp.int32, (_B, _ROWS), 0))
        u_scr[...] += jnp.dot(sel.astype(jnp.float32), y,
                              preferred_element_type=jnp.float32) * (1.0 / _H)

    @pl.when(g == _NB - 1)
    def _():
        u = u_scr[...]
        c3 = c_scr[...].reshape(_B, _S, _D)
        out_ref[...] = jnp.sum(c3 * u[:, None, :], axis=-1)


def _dense_call(t_raw, node_e, agg, W_c1, b_c1, W_c2, b_c2,
                W_ae, b_ae, W_a1, b_a1, W_a2, W_m1, b_m1, W_m2, b_m2):
    full2 = lambda arr: pl.BlockSpec(arr.shape, lambda g: (0,) * arr.ndim)
    return pl.pallas_call(
        _dense_body,
        grid=(_NB,),
        in_specs=[
            pl.BlockSpec((_ROWS, 768), lambda g: (g, 0)),
            pl.BlockSpec((_ROWS * _T, _D), lambda g: (g, 0)),
            pl.BlockSpec((_ROWS * _T, _D), lambda g: (g, 0)),
            full2(W_c1), full2(b_c1), full2(W_c2), full2(b_c2),
            full2(W_ae), full2(b_ae), full2(W_a1), full2(b_a1), full2(W_a2),
            full2(W_m1), full2(b_m1), full2(W_m2), full2(b_m2),
        ],
        out_specs=pl.BlockSpec((_B, _S), lambda g: (0, 0)),
        out_shape=jax.ShapeDtypeStruct((_B, _S), jnp.float32),
        scratch_shapes=[
            pltpu.VMEM((_CBLK * _ROWS, _D), jnp.float32),
            pltpu.VMEM((_B, _D), jnp.float32),
        ],
    )(t_raw, node_e, agg, W_c1, b_c1, W_c2, b_c2,
      W_ae, b_ae, W_a1, b_a1, W_a2, W_m1, b_m1, W_m2, b_m2)


def kernel(cand_news, clicked_news, cand_anchor_graph1, clicked_anchor_graph2,
           entity_adj, relation_adj, news_title_embedding, entity_embedding,
           relation_embedding, W_c1, b_c1, W_c2, b_c2, W_m1, b_m1, W_m2, b_m2,
           W_ae, b_ae, W_a1, b_a1, W_a2, b_a2):
    del b_a2

    news_flat = jnp.concatenate([cand_news.reshape(-1),
                                 clicked_news.reshape(-1)])
    nodes_flat = jnp.concatenate([cand_anchor_graph1.reshape(-1),
                                  clicked_anchor_graph2.reshape(-1)])

    t_raw = jnp.take(news_title_embedding, news_flat, axis=0)
    nodes_pad = jnp.pad(nodes_flat, (0, _N_PAD - _N_NODES))
    flate = jnp.take(entity_adj, nodes_pad, axis=0).reshape(-1)
    flatr = jnp.take(relation_adj, nodes_pad, axis=0).reshape(-1)
    node_e, agg = _sc_gather(nodes_pad, flate, flatr,
                             entity_embedding, relation_embedding)

    return _dense_call(t_raw, node_e, agg, W_c1, b_c1, W_c2, b_c2,
                       W_ae, b_ae, W_a1, b_a1, W_a2, W_m1, b_m1, W_m2, b_m2)


# speedup vs baseline: 1.0465x; 1.0465x over previous
import functools

import jax
import jax.numpy as jnp
from jax import lax
from jax.experimental import pallas as pl
from jax.experimental.pallas import tpu as pltpu
from jax.experimental.pallas import tpu_sc as plsc

_D = 128
_T = 20
_B = 32
_S = 5
_H = 50
_ROWS = 80
_NB = 1760 // _ROWS
_CBLK = 160 // _ROWS

_K = 10
_NW = 32
_N_NODES = 35200
_WPN = 1104
_N_PAD = _NW * _WPN
_C = 24
_SUB = _WPN // _C


def _elu(x):
    return jnp.where(x > 0, x, jnp.exp(x) - 1.0)


def _sc_body(nodes_hbm, flate_hbm, flatr_hbm, ent_hbm, rel_hbm,
             node_out, agg_out,
             idx_v, flate_v, flatr_v, node_a, node_b, nb_v, agg_a, agg_b,
             rel_v, sem0, sem1, sem2, semw0, semw1):
    wid = lax.axis_index("s") * 2 + lax.axis_index("c")
    slab = wid * _WPN
    pltpu.sync_copy(rel_hbm, rel_v)
    pltpu.sync_copy(nodes_hbm.at[pl.ds(slab, _WPN)], idx_v)
    pltpu.sync_copy(flate_hbm.at[pl.ds(slab * _K, _WPN * _K)], flate_v)
    pltpu.sync_copy(flatr_hbm.at[pl.ds(slab * _K, _WPN * _K)],
                    flatr_v.at[pl.ds(0, _WPN * _K)])

    _HC = _C // 2
    _HR = _HC * _K

    def start_half(g, buf):
        return pltpu.async_copy(
            ent_hbm.at[flate_v.at[pl.ds(g * _HR, _HR)]],
            nb_v.at[pl.ds(buf * _HR, _HR)], sem0 if buf == 0 else sem1)

    def wait_half(buf):
        pltpu.make_async_copy(
            ent_hbm.at[flate_v.at[pl.ds(0, _HR)]],
            nb_v.at[pl.ds(buf * _HR, _HR)], sem0 if buf == 0 else sem1).wait()

    def drain_writes(node_buf, agg_buf, semw):
        pltpu.make_async_copy(node_buf, node_out.at[pl.ds(0, _C)],
                              semw).wait()
        pltpu.make_async_copy(agg_buf, agg_out.at[pl.ds(0, _C)],
                              semw).wait()

    def red_half(cbase, h, agg_buf):
        def body(lc, carry2):
            c2 = h * _HC + lc
            v = flatr_v[pl.ds((cbase + c2) * _K, 16)]
            rows = [v[k] for k in range(_K)]
            nbb = h * _HR + lc * _K
            for col in range(_D // 16):
                sl = pl.ds(col * 16, 16)
                acc = nb_v[nbb, sl]
                for k in range(1, _K):
                    acc = acc + nb_v[nbb + k, sl]
                for k in range(_K):
                    acc = acc + rel_v[rows[k], sl]
                agg_buf[c2, sl] = acc
            return carry2
        lax.fori_loop(0, _HC, body, 0)

    start_half(0, 0)

    def pair(p, carry):
        for par in (0, 1):
            s = 2 * p + par
            cbase = s * _C
            node_buf = node_a if par == 0 else node_b
            agg_buf = agg_a if par == 0 else agg_b
            semw = semw0 if par == 0 else semw1
            start_half(2 * s + 1, 1)

            @pl.when(p > 0)
            def _():
                drain_writes(node_buf, agg_buf, semw)

            wait_half(0)
            cp_n = pltpu.async_copy(
                ent_hbm.at[idx_v.at[pl.ds(cbase, _C)]], node_buf, sem2)
            red_half(cbase, 0, agg_buf)

            @pl.when(s < _SUB - 1)
            def _():
                start_half(2 * s + 2, 0)

            wait_half(1)
            red_half(cbase, 1, agg_buf)
            cp_n.wait()
            pltpu.async_copy(node_buf,
                             node_out.at[pl.ds(slab + cbase, _C)], semw)
            pltpu.async_copy(agg_buf,
                             agg_out.at[pl.ds(slab + cbase, _C)], semw)
        return carry

    lax.fori_loop(0, _SUB // 2, pair, 0)
    drain_writes(node_a, agg_a, semw0)
    drain_writes(node_b, agg_b, semw1)


def _sc_gather(nodes_pad, flate, flatr, ent_emb, rel_emb):
    mesh = plsc.VectorSubcoreMesh(core_axis_name="c", subcore_axis_name="s")
    f32, i32 = jnp.float32, jnp.int32
    return pl.kernel(
        _sc_body,
        out_type=(jax.ShapeDtypeStruct((_N_PAD, _D), f32),
                  jax.ShapeDtypeStruct((_N_PAD, _D), f32)),
        mesh=mesh,
        scratch_types=[
            pltpu.VMEM((_WPN,), i32),
            pltpu.VMEM((_WPN * _K,), i32),
            pltpu.VMEM((_WPN * _K + 16,), i32),
            pltpu.VMEM((_C, _D), f32),
            pltpu.VMEM((_C, _D), f32),
            pltpu.VMEM((_C * _K, _D), f32),
            pltpu.VMEM((_C, _D), f32),
            pltpu.VMEM((_C, _D), f32),
            pltpu.VMEM((500, _D), f32),
            pltpu.SemaphoreType.DMA,
            pltpu.SemaphoreType.DMA,
            pltpu.SemaphoreType.DMA,
            pltpu.SemaphoreType.DMA,
            pltpu.SemaphoreType.DMA,
        ],
    )(nodes_pad, flate, flatr, ent_emb, rel_emb)


def _dense_body(t_raw_ref, node_ref, agg_ref,
                W_c1_ref, b_c1_ref, W_c2_ref, b_c2_ref,
                W_ae_ref, b_ae_ref, W_a1_ref, b_a1_ref, W_a2_ref,
                W_m1_ref, b_m1_ref, W_m2_ref, b_m2_ref,
                out_ref, c_scr, u_scr):
    g = pl.program_id(0)

    t = t_raw_ref[...]
    t = _elu(jnp.dot(t, W_c1_ref[...], preferred_element_type=jnp.float32)
             + b_c1_ref[...])
    t = jnp.tanh(jnp.dot(t, W_c2_ref[...], preferred_element_type=jnp.float32)
                 + b_c2_ref[...])

    node = node_ref[...]
    agg = agg_ref[...]
    W_ae = W_ae_ref[...]
    a = jnp.tanh(jnp.dot(node, W_ae[:_D], preferred_element_type=jnp.float32)
                 + jnp.dot(agg, W_ae[_D:], preferred_element_type=jnp.float32)
                 + b_ae_ref[...])
    h = _elu(jnp.dot(a, W_a1_ref[...], preferred_element_type=jnp.float32)
             + b_a1_ref[...])
    w2b = jnp.broadcast_to(W_a2_ref[...], (_D, _D))
    lg = jnp.dot(h, w2b, preferred_element_type=jnp.float32)
    ex = jnp.exp(jnp.minimum(lg, 60.0))
    colg = lax.broadcasted_iota(jnp.int32, (_ROWS, _ROWS * _T), 1) // _T
    rowg = lax.broadcasted_iota(jnp.int32, (_ROWS, _ROWS * _T), 0)
    p_sum = (colg == rowg).astype(jnp.float32)
    rowg2 = lax.broadcasted_iota(jnp.int32, (_ROWS * _T, _ROWS), 0) // _T
    colg2 = lax.broadcasted_iota(jnp.int32, (_ROWS * _T, _ROWS), 1)
    p_exp = (rowg2 == colg2).astype(jnp.float32)
    s = jnp.dot(p_sum, ex, preferred_element_type=jnp.float32)
    sfull = jnp.dot(p_exp, s, preferred_element_type=jnp.float32)
    wfull = ex / (sfull + 1e-30)
    anchor = jnp.dot(p_sum, a * wfull,
                     preferred_element_type=jnp.float32)

    W_m1 = W_m1_ref[...]
    y = _elu(jnp.dot(t, W_m1[:_D], preferred_element_type=jnp.float32)
             + jnp.dot(anchor, W_m1[_D:], preferred_element_type=jnp.float32)
             + b_m1_ref[...])
    y = _elu(jnp.dot(y, W_m2_ref[...], preferred_element_type=jnp.float32)
             + b_m2_ref[...])

    @pl.when(g == 0)
    def _():
        u_scr[...] = jnp.zeros_like(u_scr)

    @pl.when(g < _CBLK)
    def _():
        c_scr[pl.ds(g * _ROWS, _ROWS), :] = y

    @pl.when(g >= _CBLK)
    def _():
        rows = (g - _CBLK) * _ROWS + lax.broadcasted_iota(jnp.int32, (_B, _ROWS), 1)
        sel = (rows // _H == lax.broadcasted_iota(jnp.int32, (_B, _ROWS), 0))
        u_scr[...] += jnp.dot(sel.astype(jnp.float32), y,
                              preferred_element_type=jnp.float32) * (1.0 / _H)

    @pl.when(g == _NB - 1)
    def _():
        u = u_scr[...]
        c3 = c_scr[...].reshape(_B, _S, _D)
        out_ref[...] = jnp.sum(c3 * u[:, None, :], axis=-1)


def _dense_call(t_raw, node_e, agg, W_c1, b_c1, W_c2, b_c2,
                W_ae, b_ae, W_a1, b_a1, W_a2, W_m1, b_m1, W_m2, b_m2):
    full2 = lambda arr: pl.BlockSpec(arr.shape, lambda g: (0,) * arr.ndim)
    return pl.pallas_call(
        _dense_body,
        grid=(_NB,),
        in_specs=[
            pl.BlockSpec((_ROWS, 768), lambda g: (g, 0)),
            pl.BlockSpec((_ROWS * _T, _D), lambda g: (g, 0)),
            pl.BlockSpec((_ROWS * _T, _D), lambda g: (g, 0)),
            full2(W_c1), full2(b_c1), full2(W_c2), full2(b_c2),
            full2(W_ae), full2(b_ae), full2(W_a1), full2(b_a1), full2(W_a2),
            full2(W_m1), full2(b_m1), full2(W_m2), full2(b_m2),
        ],
        out_specs=pl.BlockSpec((_B, _S), lambda g: (0, 0)),
        out_shape=jax.ShapeDtypeStruct((_B, _S), jnp.float32),
        scratch_shapes=[
            pltpu.VMEM((_CBLK * _ROWS, _D), jnp.float32),
            pltpu.VMEM((_B, _D), jnp.float32),
        ],
    )(t_raw, node_e, agg, W_c1, b_c1, W_c2, b_c2,
      W_ae, b_ae, W_a1, b_a1, W_a2, W_m1, b_m1, W_m2, b_m2)


def kernel(cand_news, clicked_news, cand_anchor_graph1, clicked_anchor_graph2,
           entity_adj, relation_adj, news_title_embedding, entity_embedding,
           relation_embedding, W_c1, b_c1, W_c2, b_c2, W_m1, b_m1, W_m2, b_m2,
           W_ae, b_ae, W_a1, b_a1, W_a2, b_a2):
    del b_a2

    news_flat = jnp.concatenate([cand_news.reshape(-1),
                                 clicked_news.reshape(-1)])
    nodes_flat = jnp.concatenate([cand_anchor_graph1.reshape(-1),
                                  clicked_anchor_graph2.reshape(-1)])

    t_raw = jnp.take(news_title_embedding, news_flat, axis=0)
    nodes_pad = jnp.pad(nodes_flat, (0, _N_PAD - _N_NODES))
    flate = jnp.take(entity_adj, nodes_pad, axis=0).reshape(-1)
    flatr = jnp.take(relation_adj, nodes_pad, axis=0).reshape(-1)
    node_e, agg = _sc_gather(nodes_pad, flate, flatr,
                             entity_embedding, relation_embedding)

    return _dense_call(t_raw, node_e, agg, W_c1, b_c1, W_c2, b_c2,
                       W_ae, b_ae, W_a1, b_a1, W_a2, W_m1, b_m1, W_m2, b_m2)


# speedup vs baseline: 1.0604x; 1.0133x over previous
import functools

import jax
import jax.numpy as jnp
from jax import lax
from jax.experimental import pallas as pl
from jax.experimental.pallas import tpu as pltpu
from jax.experimental.pallas import tpu_sc as plsc

_D = 128
_T = 20
_B = 32
_S = 5
_H = 50
_ROWS = 160
_NB = 1760 // _ROWS
_CBLK = 160 // _ROWS

_K = 10
_NW = 32
_N_NODES = 35200
_WPN = 1104
_N_PAD = _NW * _WPN
_C = 24
_SUB = _WPN // _C


def _elu(x):
    return jnp.where(x > 0, x, jnp.exp(x) - 1.0)


def _sc_body(nodes_hbm, flate_hbm, flatr_hbm, ent_hbm, rel_hbm,
             node_out, agg_out,
             idx_v, flate_v, flatr_v, node_a, node_b, nb_v, agg_a, agg_b,
             rel_v, sem0, sem1, sem2, semw0, semw1):
    wid = lax.axis_index("s") * 2 + lax.axis_index("c")
    slab = wid * _WPN
    pltpu.sync_copy(rel_hbm, rel_v)
    pltpu.sync_copy(nodes_hbm.at[pl.ds(slab, _WPN)], idx_v)
    pltpu.sync_copy(flate_hbm.at[pl.ds(slab * _K, _WPN * _K)], flate_v)
    pltpu.sync_copy(flatr_hbm.at[pl.ds(slab * _K, _WPN * _K)],
                    flatr_v.at[pl.ds(0, _WPN * _K)])

    _HC = _C // 2
    _HR = _HC * _K

    def start_half(g, buf):
        return pltpu.async_copy(
            ent_hbm.at[flate_v.at[pl.ds(g * _HR, _HR)]],
            nb_v.at[pl.ds(buf * _HR, _HR)], sem0 if buf == 0 else sem1)

    def wait_half(buf):
        pltpu.make_async_copy(
            ent_hbm.at[flate_v.at[pl.ds(0, _HR)]],
            nb_v.at[pl.ds(buf * _HR, _HR)], sem0 if buf == 0 else sem1).wait()

    def drain_writes(node_buf, agg_buf, semw):
        pltpu.make_async_copy(node_buf, node_out.at[pl.ds(0, _C)],
                              semw).wait()
        pltpu.make_async_copy(agg_buf, agg_out.at[pl.ds(0, _C)],
                              semw).wait()

    def red_half(cbase, h, agg_buf):
        def body(lc, carry2):
            c2 = h * _HC + lc
            v = flatr_v[pl.ds((cbase + c2) * _K, 16)]
            rows = [v[k] for k in range(_K)]
            nbb = h * _HR + lc * _K
            for col in range(_D // 16):
                sl = pl.ds(col * 16, 16)
                acc = nb_v[nbb, sl]
                for k in range(1, _K):
                    acc = acc + nb_v[nbb + k, sl]
                for k in range(_K):
                    acc = acc + rel_v[rows[k], sl]
                agg_buf[c2, sl] = acc
            return carry2
        lax.fori_loop(0, _HC, body, 0)

    start_half(0, 0)

    def pair(p, carry):
        for par in (0, 1):
            s = 2 * p + par
            cbase = s * _C
            node_buf = node_a if par == 0 else node_b
            agg_buf = agg_a if par == 0 else agg_b
            semw = semw0 if par == 0 else semw1
            start_half(2 * s + 1, 1)

            @pl.when(p > 0)
            def _():
                drain_writes(node_buf, agg_buf, semw)

            wait_half(0)
            cp_n = pltpu.async_copy(
                ent_hbm.at[idx_v.at[pl.ds(cbase, _C)]], node_buf, sem2)
            red_half(cbase, 0, agg_buf)

            @pl.when(s < _SUB - 1)
            def _():
                start_half(2 * s + 2, 0)

            wait_half(1)
            red_half(cbase, 1, agg_buf)
            cp_n.wait()
            pltpu.async_copy(node_buf,
                             node_out.at[pl.ds(slab + cbase, _C)], semw)
            pltpu.async_copy(agg_buf,
                             agg_out.at[pl.ds(slab + cbase, _C)], semw)
        return carry

    lax.fori_loop(0, _SUB // 2, pair, 0)
    drain_writes(node_a, agg_a, semw0)
    drain_writes(node_b, agg_b, semw1)


def _sc_gather(nodes_pad, flate, flatr, ent_emb, rel_emb):
    mesh = plsc.VectorSubcoreMesh(core_axis_name="c", subcore_axis_name="s")
    f32, i32 = jnp.float32, jnp.int32
    return pl.kernel(
        _sc_body,
        out_type=(jax.ShapeDtypeStruct((_N_PAD, _D), f32),
                  jax.ShapeDtypeStruct((_N_PAD, _D), f32)),
        mesh=mesh,
        scratch_types=[
            pltpu.VMEM((_WPN,), i32),
            pltpu.VMEM((_WPN * _K,), i32),
            pltpu.VMEM((_WPN * _K + 16,), i32),
            pltpu.VMEM((_C, _D), f32),
            pltpu.VMEM((_C, _D), f32),
            pltpu.VMEM((_C * _K, _D), f32),
            pltpu.VMEM((_C, _D), f32),
            pltpu.VMEM((_C, _D), f32),
            pltpu.VMEM((500, _D), f32),
            pltpu.SemaphoreType.DMA,
            pltpu.SemaphoreType.DMA,
            pltpu.SemaphoreType.DMA,
            pltpu.SemaphoreType.DMA,
            pltpu.SemaphoreType.DMA,
        ],
    )(nodes_pad, flate, flatr, ent_emb, rel_emb)


def _dense_body(t_raw_ref, node_ref, agg_ref,
                W_c1_ref, b_c1_ref, W_c2_ref, b_c2_ref,
                W_ae_ref, b_ae_ref, W_a1_ref, b_a1_ref, W_a2_ref,
                W_m1_ref, b_m1_ref, W_m2_ref, b_m2_ref,
                out_ref, c_scr, u_scr):
    g = pl.program_id(0)

    t = t_raw_ref[...]
    t = _elu(jnp.dot(t, W_c1_ref[...], preferred_element_type=jnp.float32)
             + b_c1_ref[...])
    t = jnp.tanh(jnp.dot(t, W_c2_ref[...], preferred_element_type=jnp.float32)
                 + b_c2_ref[...])

    node = node_ref[...]
    agg = agg_ref[...]
    W_ae = W_ae_ref[...]
    a = jnp.tanh(jnp.dot(node, W_ae[:_D], preferred_element_type=jnp.float32)
                 + jnp.dot(agg, W_ae[_D:], preferred_element_type=jnp.float32)
                 + b_ae_ref[...])
    h = _elu(jnp.dot(a, W_a1_ref[...], preferred_element_type=jnp.float32)
             + b_a1_ref[...])
    w2b = jnp.broadcast_to(W_a2_ref[...], (_D, _D))
    lg = jnp.dot(h, w2b, preferred_element_type=jnp.float32)
    ex = jnp.exp(jnp.minimum(lg, 60.0))
    colg = lax.broadcasted_iota(jnp.int32, (_ROWS, _ROWS * _T), 1) // _T
    rowg = lax.broadcasted_iota(jnp.int32, (_ROWS, _ROWS * _T), 0)
    p_sum = (colg == rowg).astype(jnp.float32)
    rowg2 = lax.broadcasted_iota(jnp.int32, (_ROWS * _T, _ROWS), 0) // _T
    colg2 = lax.broadcasted_iota(jnp.int32, (_ROWS * _T, _ROWS), 1)
    p_exp = (rowg2 == colg2).astype(jnp.float32)
    s = jnp.dot(p_sum, ex, preferred_element_type=jnp.float32)
    sfull = jnp.dot(p_exp, s, preferred_element_type=jnp.float32)
    wfull = ex / (sfull + 1e-30)
    anchor = jnp.dot(p_sum, a * wfull,
                     preferred_element_type=jnp.float32)

    W_m1 = W_m1_ref[...]
    y = _elu(jnp.dot(t, W_m1[:_D], preferred_element_type=jnp.float32)
             + jnp.dot(anchor, W_m1[_D:], preferred_element_type=jnp.float32)
             + b_m1_ref[...])
    y = _elu(jnp.dot(y, W_m2_ref[...], preferred_element_type=jnp.float32)
             + b_m2_ref[...])

    @pl.when(g == 0)
    def _():
        u_scr[...] = jnp.zeros_like(u_scr)

    @pl.when(g < _CBLK)
    def _():
        c_scr[pl.ds(g * _ROWS, _ROWS), :] = y

    @pl.when(g >= _CBLK)
    def _():
        rows = (g - _CBLK) * _ROWS + lax.broadcasted_iota(jnp.int32, (_B, _ROWS), 1)
        sel = (rows // _H == lax.broadcasted_iota(jnp.int32, (_B, _ROWS), 0))
        u_scr[...] += jnp.dot(sel.astype(jnp.float32), y,
                              preferred_element_type=jnp.float32) * (1.0 / _H)

    @pl.when(g == _NB - 1)
    def _():
        u = u_scr[...]
        c3 = c_scr[...].reshape(_B, _S, _D)
        out_ref[...] = jnp.sum(c3 * u[:, None, :], axis=-1)


def _dense_call(t_raw, node_e, agg, W_c1, b_c1, W_c2, b_c2,
                W_ae, b_ae, W_a1, b_a1, W_a2, W_m1, b_m1, W_m2, b_m2):
    full2 = lambda arr: pl.BlockSpec(arr.shape, lambda g: (0,) * arr.ndim)
    return pl.pallas_call(
        _dense_body,
        grid=(_NB,),
        in_specs=[
            pl.BlockSpec((_ROWS, 768), lambda g: (g, 0)),
            pl.BlockSpec((_ROWS * _T, _D), lambda g: (g, 0)),
            pl.BlockSpec((_ROWS * _T, _D), lambda g: (g, 0)),
            full2(W_c1), full2(b_c1), full2(W_c2), full2(b_c2),
            full2(W_ae), full2(b_ae), full2(W_a1), full2(b_a1), full2(W_a2),
            full2(W_m1), full2(b_m1), full2(W_m2), full2(b_m2),
        ],
        out_specs=pl.BlockSpec((_B, _S), lambda g: (0, 0)),
        out_shape=jax.ShapeDtypeStruct((_B, _S), jnp.float32),
        scratch_shapes=[
            pltpu.VMEM((_CBLK * _ROWS, _D), jnp.float32),
            pltpu.VMEM((_B, _D), jnp.float32),
        ],
    )(t_raw, node_e, agg, W_c1, b_c1, W_c2, b_c2,
      W_ae, b_ae, W_a1, b_a1, W_a2, W_m1, b_m1, W_m2, b_m2)


def kernel(cand_news, clicked_news, cand_anchor_graph1, clicked_anchor_graph2,
           entity_adj, relation_adj, news_title_embedding, entity_embedding,
           relation_embedding, W_c1, b_c1, W_c2, b_c2, W_m1, b_m1, W_m2, b_m2,
           W_ae, b_ae, W_a1, b_a1, W_a2, b_a2):
    del b_a2

    news_flat = jnp.concatenate([cand_news.reshape(-1),
                                 clicked_news.reshape(-1)])
    nodes_flat = jnp.concatenate([cand_anchor_graph1.reshape(-1),
                                  clicked_anchor_graph2.reshape(-1)])

    t_raw = jnp.take(news_title_embedding, news_flat, axis=0)
    nodes_pad = jnp.pad(nodes_flat, (0, _N_PAD - _N_NODES))
    flate = jnp.take(entity_adj, nodes_pad, axis=0).reshape(-1)
    flatr = jnp.take(relation_adj, nodes_pad, axis=0).reshape(-1)
    node_e, agg = _sc_gather(nodes_pad, flate, flatr,
                             entity_embedding, relation_embedding)

    return _dense_call(t_raw, node_e, agg, W_c1, b_c1, W_c2, b_c2,
                       W_ae, b_ae, W_a1, b_a1, W_a2, W_m1, b_m1, W_m2, b_m2)


# speedup vs baseline: 1.0641x; 1.0035x over previous
import functools

import jax
import jax.numpy as jnp
from jax import lax
from jax.experimental import pallas as pl
from jax.experimental.pallas import tpu as pltpu
from jax.experimental.pallas import tpu_sc as plsc

_D = 128
_T = 20
_B = 32
_S = 5
_H = 50
_ROWS = 160
_NB = 1760 // _ROWS
_CBLK = 160 // _ROWS

_K = 10
_NW = 32
_N_NODES = 35200
_WPN = 1104
_N_PAD = _NW * _WPN
_C = 24
_SUB = _WPN // _C


def _elu(x):
    return jnp.where(x > 0, x, jnp.exp(x) - 1.0)


def _sc_body(nodes_hbm, flate_hbm, flatr_hbm, ent_hbm, rel_hbm,
             node_out, agg_out,
             idx_v, flate_v, flatr_v, node_a, node_b, nb_v, agg_a, agg_b,
             rel_v, sem0, sem1, sem2, semw0, semw1):
    wid = lax.axis_index("s") * 2 + lax.axis_index("c")
    slab = wid * _WPN
    pltpu.sync_copy(rel_hbm, rel_v)
    pltpu.sync_copy(nodes_hbm.at[pl.ds(slab, _WPN)], idx_v)
    pltpu.sync_copy(flate_hbm.at[pl.ds(slab * _K, _WPN * _K)], flate_v)
    pltpu.sync_copy(flatr_hbm.at[pl.ds(slab * _K, _WPN * _K)],
                    flatr_v.at[pl.ds(0, _WPN * _K)])

    _HC = _C // 2
    _HR = _HC * _K

    def start_half(g, buf):
        return pltpu.async_copy(
            ent_hbm.at[flate_v.at[pl.ds(g * _HR, _HR)]],
            nb_v.at[pl.ds(buf * _HR, _HR)], sem0 if buf == 0 else sem1)

    def wait_half(buf):
        pltpu.make_async_copy(
            ent_hbm.at[flate_v.at[pl.ds(0, _HR)]],
            nb_v.at[pl.ds(buf * _HR, _HR)], sem0 if buf == 0 else sem1).wait()

    def drain_writes(node_buf, agg_buf, semw):
        pltpu.make_async_copy(node_buf, node_out.at[pl.ds(0, _C)],
                              semw).wait()
        pltpu.make_async_copy(agg_buf, agg_out.at[pl.ds(0, _C)],
                              semw).wait()

    def red_half(cbase, h, agg_buf):
        def body(lc, carry2):
            c2 = h * _HC + lc
            v = flatr_v[pl.ds((cbase + c2) * _K, 16)]
            rows = [v[k] for k in range(_K)]
            nbb = h * _HR + lc * _K
            for col in range(_D // 16):
                sl = pl.ds(col * 16, 16)
                acc = nb_v[nbb, sl]
                for k in range(1, _K):
                    acc = acc + nb_v[nbb + k, sl]
                for k in range(_K):
                    acc = acc + rel_v[rows[k], sl]
                agg_buf[c2, sl] = acc
            return carry2
        lax.fori_loop(0, _HC, body, 0)

    start_half(0, 0)

    def pair(p, carry):
        for par in (0, 1):
            s = 2 * p + par
            cbase = s * _C
            node_buf = node_a if par == 0 else node_b
            agg_buf = agg_a if par == 0 else agg_b
            semw = semw0 if par == 0 else semw1
            start_half(2 * s + 1, 1)

            @pl.when(p > 0)
            def _():
                drain_writes(node_buf, agg_buf, semw)

            wait_half(0)
            cp_n = pltpu.async_copy(
                ent_hbm.at[idx_v.at[pl.ds(cbase, _C)]], node_buf, sem2)
            red_half(cbase, 0, agg_buf)

            @pl.when(s < _SUB - 1)
            def _():
                start_half(2 * s + 2, 0)

            wait_half(1)
            red_half(cbase, 1, agg_buf)
            cp_n.wait()
            pltpu.async_copy(node_buf,
                             node_out.at[pl.ds(slab + cbase, _C)], semw)
            pltpu.async_copy(agg_buf,
                             agg_out.at[pl.ds(slab + cbase, _C)], semw)
        return carry

    lax.fori_loop(0, _SUB // 2, pair, 0)
    drain_writes(node_a, agg_a, semw0)
    drain_writes(node_b, agg_b, semw1)


def _sc_gather(nodes_pad, flate, flatr, ent_emb, rel_emb):
    mesh = plsc.VectorSubcoreMesh(core_axis_name="c", subcore_axis_name="s")
    f32, i32 = jnp.float32, jnp.int32
    return pl.kernel(
        _sc_body,
        out_type=(jax.ShapeDtypeStruct((_N_PAD, _D), f32),
                  jax.ShapeDtypeStruct((_N_PAD, _D), f32)),
        mesh=mesh,
        scratch_types=[
            pltpu.VMEM((_WPN,), i32),
            pltpu.VMEM((_WPN * _K,), i32),
            pltpu.VMEM((_WPN * _K + 16,), i32),
            pltpu.VMEM((_C, _D), f32),
            pltpu.VMEM((_C, _D), f32),
            pltpu.VMEM((_C * _K, _D), f32),
            pltpu.VMEM((_C, _D), f32),
            pltpu.VMEM((_C, _D), f32),
            pltpu.VMEM((500, _D), f32),
            pltpu.SemaphoreType.DMA,
            pltpu.SemaphoreType.DMA,
            pltpu.SemaphoreType.DMA,
            pltpu.SemaphoreType.DMA,
            pltpu.SemaphoreType.DMA,
        ],
    )(nodes_pad, flate, flatr, ent_emb, rel_emb)


def _title_body(t_raw_ref, W_c1_ref, b_c1_ref, W_c2_ref, b_c2_ref, out_ref):
    t = _elu(jnp.dot(t_raw_ref[...], W_c1_ref[...],
                     preferred_element_type=jnp.float32) + b_c1_ref[...])
    out_ref[...] = jnp.tanh(
        jnp.dot(t, W_c2_ref[...], preferred_element_type=jnp.float32)
        + b_c2_ref[...])


def _title_call(t_raw, W_c1, b_c1, W_c2, b_c2):
    full2 = lambda arr: pl.BlockSpec(arr.shape, lambda g: (0,) * arr.ndim)
    return pl.pallas_call(
        _title_body,
        grid=(_NB,),
        in_specs=[pl.BlockSpec((_ROWS, 768), lambda g: (g, 0)),
                  full2(W_c1), full2(b_c1), full2(W_c2), full2(b_c2)],
        out_specs=pl.BlockSpec((_ROWS, _D), lambda g: (g, 0)),
        out_shape=jax.ShapeDtypeStruct((1760, _D), jnp.float32),
    )(t_raw, W_c1, b_c1, W_c2, b_c2)


def _dense_body(t_ref, node_ref, agg_ref,
                W_ae_ref, b_ae_ref, W_a1_ref, b_a1_ref, W_a2_ref,
                W_m1_ref, b_m1_ref, W_m2_ref, b_m2_ref,
                out_ref, c_scr, u_scr):
    g = pl.program_id(0)
    t = t_ref[...]

    node = node_ref[...]
    agg = agg_ref[...]
    W_ae = W_ae_ref[...]
    a = jnp.tanh(jnp.dot(node, W_ae[:_D], preferred_element_type=jnp.float32)
                 + jnp.dot(agg, W_ae[_D:], preferred_element_type=jnp.float32)
                 + b_ae_ref[...])
    h = _elu(jnp.dot(a, W_a1_ref[...], preferred_element_type=jnp.float32)
             + b_a1_ref[...])
    w2b = jnp.broadcast_to(W_a2_ref[...], (_D, _D))
    lg = jnp.dot(h, w2b, preferred_element_type=jnp.float32)
    ex = jnp.exp(jnp.minimum(lg, 60.0))
    colg = lax.broadcasted_iota(jnp.int32, (_ROWS, _ROWS * _T), 1) // _T
    rowg = lax.broadcasted_iota(jnp.int32, (_ROWS, _ROWS * _T), 0)
    p_sum = (colg == rowg).astype(jnp.float32)
    rowg2 = lax.broadcasted_iota(jnp.int32, (_ROWS * _T, _ROWS), 0) // _T
    colg2 = lax.broadcasted_iota(jnp.int32, (_ROWS * _T, _ROWS), 1)
    p_exp = (rowg2 == colg2).astype(jnp.float32)
    s = jnp.dot(p_sum, ex, preferred_element_type=jnp.float32)
    sfull = jnp.dot(p_exp, s, preferred_element_type=jnp.float32)
    wfull = ex / (sfull + 1e-30)
    anchor = jnp.dot(p_sum, a * wfull,
                     preferred_element_type=jnp.float32)

    W_m1 = W_m1_ref[...]
    y = _elu(jnp.dot(t, W_m1[:_D], preferred_element_type=jnp.float32)
             + jnp.dot(anchor, W_m1[_D:], preferred_element_type=jnp.float32)
             + b_m1_ref[...])
    y = _elu(jnp.dot(y, W_m2_ref[...], preferred_element_type=jnp.float32)
             + b_m2_ref[...])

    @pl.when(g == 0)
    def _():
        u_scr[...] = jnp.zeros_like(u_scr)

    @pl.when(g < _CBLK)
    def _():
        c_scr[pl.ds(g * _ROWS, _ROWS), :] = y

    @pl.when(g >= _CBLK)
    def _():
        rows = (g - _CBLK) * _ROWS + lax.broadcasted_iota(jnp.int32, (_B, _ROWS), 1)
        sel = (rows // _H == lax.broadcasted_iota(jnp.int32, (_B, _ROWS), 0))
        u_scr[...] += jnp.dot(sel.astype(jnp.float32), y,
                              preferred_element_type=jnp.float32) * (1.0 / _H)

    @pl.when(g == _NB - 1)
    def _():
        u = u_scr[...]
        c3 = c_scr[...].reshape(_B, _S, _D)
        out_ref[...] = jnp.sum(c3 * u[:, None, :], axis=-1)


def _dense_call(t, node_e, agg,
                W_ae, b_ae, W_a1, b_a1, W_a2, W_m1, b_m1, W_m2, b_m2):
    full2 = lambda arr: pl.BlockSpec(arr.shape, lambda g: (0,) * arr.ndim)
    return pl.pallas_call(
        _dense_body,
        grid=(_NB,),
        in_specs=[
            pl.BlockSpec((_ROWS, _D), lambda g: (g, 0)),
            pl.BlockSpec((_ROWS * _T, _D), lambda g: (g, 0)),
            pl.BlockSpec((_ROWS * _T, _D), lambda g: (g, 0)),
            full2(W_ae), full2(b_ae), full2(W_a1), full2(b_a1), full2(W_a2),
            full2(W_m1), full2(b_m1), full2(W_m2), full2(b_m2),
        ],
        out_specs=pl.BlockSpec((_B, _S), lambda g: (0, 0)),
        out_shape=jax.ShapeDtypeStruct((_B, _S), jnp.float32),
        scratch_shapes=[
            pltpu.VMEM((_CBLK * _ROWS, _D), jnp.float32),
            pltpu.VMEM((_B, _D), jnp.float32),
        ],
    )(t, node_e, agg,
      W_ae, b_ae, W_a1, b_a1, W_a2, W_m1, b_m1, W_m2, b_m2)


def kernel(cand_news, clicked_news, cand_anchor_graph1, clicked_anchor_graph2,
           entity_adj, relation_adj, news_title_embedding, entity_embedding,
           relation_embedding, W_c1, b_c1, W_c2, b_c2, W_m1, b_m1, W_m2, b_m2,
           W_ae, b_ae, W_a1, b_a1, W_a2, b_a2):
    del b_a2

    news_flat = jnp.concatenate([cand_news.reshape(-1),
                                 clicked_news.reshape(-1)])
    nodes_flat = jnp.concatenate([cand_anchor_graph1.reshape(-1),
                                  clicked_anchor_graph2.reshape(-1)])

    t_raw = jnp.take(news_title_embedding, news_flat, axis=0)
    nodes_pad = jnp.pad(nodes_flat, (0, _N_PAD - _N_NODES))
    flate = jnp.take(entity_adj, nodes_pad, axis=0).reshape(-1)
    flatr = jnp.take(relation_adj, nodes_pad, axis=0).reshape(-1)
    node_e, agg = _sc_gather(nodes_pad, flate, flatr,
                             entity_embedding, relation_embedding)

    t = _title_call(t_raw, W_c1, b_c1, W_c2, b_c2)
    return _dense_call(t, node_e, agg,
                       W_ae, b_ae, W_a1, b_a1, W_a2, W_m1, b_m1, W_m2, b_m2)
